# baseline (speedup 1.0000x reference)
import functools

import jax
import jax.numpy as jnp
from jax.experimental import pallas as pl
from jax.experimental.pallas import tpu as pltpu
from jax.experimental.pallas import tpu_sc as plsc

B, N1, N2 = 8, 8192, 2048
C1, C2 = 128, 256
IN_CH, OUT_CH = C1 + C2, 256
EPS_BN = 1e-5

TILE = 2048
NT = N1 // TILE

NW = 32
Q = B * N1
QW = Q // NW
CH = 32
NCHUNK = QW // CH


def _passA_body(xyz1_ref, xyz2_ref, idx_ref, w3_ref):
    b = pl.program_id(0)

    q = xyz1_ref[0]
    p = xyz2_ref[0]
    q2 = jnp.sum(q * q, axis=1, keepdims=True)
    p2 = jnp.sum(p * p, axis=1)
    qp = jax.lax.dot_general(q, p, (((1,), (1,)), ((), ())),
                             preferred_element_type=jnp.float32)
    d2 = q2 + p2[None, :] - 2.0 * qp

    iota_f = jax.lax.broadcasted_iota(jnp.int32, (TILE, N2), 1).astype(jnp.float32)
    work = d2
    idxs, ws = [], []
    w_sum = jnp.zeros((TILE, 1), jnp.float32)
    for _ in range(3):
        mk = jnp.min(work, axis=1, keepdims=True)
        sel = work == mk
        idxf = jnp.min(jnp.where(sel, iota_f, jnp.float32(1e9)), axis=1,
                       keepdims=True)
        dk = jnp.sqrt(jnp.maximum(mk, 1e-12)) + 1e-8
        wk = 1.0 / dk
        idxs.append(idxf)
        ws.append(wk)
        w_sum = w_sum + wk
        work = jnp.where(sel, jnp.float32(3.4e38), work)
    inv = 1.0 / w_sum
    idx3 = jnp.concatenate(idxs, axis=1).astype(jnp.int32) + b * N2
    w3 = jnp.concatenate([w * inv for w in ws], axis=1)
    idx_ref[0] = idx3
    w3_ref[0] = jnp.broadcast_to(w3[:, :, None], (TILE, 3, 16))


def _sc_interp_body(table_ref, idx_ref, w_ref, out_ref,
                    idx_v, w_v, rows_v, acc_v, sem):
    wid = jax.lax.axis_index("s") * 2 + jax.lax.axis_index("c")
    base = wid * QW

    def chunk(ci, carry):
        q0 = base + ci * CH
        pltpu.sync_copy(idx_ref.at[pl.ds(q0 * 3, CH * 3)], idx_v)
        pltpu.sync_copy(w_ref.at[pl.ds(q0 * 3, CH * 3)], w_v)
        pltpu.async_copy(table_ref.at[idx_v], rows_v, sem).wait()

        def qbody(qi, c2):
            r0 = qi * 3
            w0 = w_v[r0, :]
            w1 = w_v[r0 + 1, :]
            w2 = w_v[r0 + 2, :]
            for cg in range(C2 // 16):
                sl = pl.ds(cg * 16, 16)
                acc_v[qi, sl] = (rows_v[r0, sl] * w0 + rows_v[r0 + 1, sl] * w1
                                 + rows_v[r0 + 2, sl] * w2)
            return c2

        jax.lax.fori_loop(0, CH, qbody, 0)
        pltpu.sync_copy(acc_v, out_ref.at[pl.ds(q0, CH)])
        return carry

    jax.lax.fori_loop(0, NCHUNK, chunk, 0)


def _passB_body(interp_ref, feats1_ref, wi_ref, wf_ref, y_ref, partial_ref):
    b = pl.program_id(0)
    n = pl.program_id(1)
    it = interp_ref[0]
    y = (jax.lax.dot_general(wi_ref[...], it, (((1,), (1,)), ((), ())),
                             preferred_element_type=jnp.float32)
         + jax.lax.dot_general(wf_ref[...], feats1_ref[0],
                               (((1,), (0,)), ((), ())),
                               preferred_element_type=jnp.float32))
    y_ref[0] = y.astype(jnp.bfloat16)

    ps = jnp.sum(y, axis=1)
    pss = jnp.sum(y * y, axis=1)
    part = jnp.stack([ps, pss], axis=0)

    @pl.when(jnp.logical_and(b == 0, n == 0))
    def _init():
        partial_ref[...] = part

    @pl.when(jnp.logical_or(b != 0, n != 0))
    def _acc():
        partial_ref[...] = partial_ref[...] + part


def _pass2_body(y_ref, partial_ref, gamma_ref, beta_ref, out_ref):
    sums = partial_ref[...]
    cnt = jnp.float32(B * N1)
    mean = sums[0] / cnt
    var = sums[1] / cnt - mean * mean
    scale = gamma_ref[...][0] / jnp.sqrt(var + EPS_BN)
    shift = beta_ref[...][0] - mean * scale
    y = y_ref[0].astype(jnp.float32)
    out_ref[0] = jnp.maximum(y * scale[:, None] + shift[:, None], 0.0)


@jax.jit
def kernel(xyz1, xyz2, feats1, feats2, W, gamma, beta):
    idx3, w3 = pl.pallas_call(
        _passA_body,
        grid=(B, NT),
        in_specs=[
            pl.BlockSpec((1, TILE, 3), lambda b, n: (b, n, 0)),
            pl.BlockSpec((1, N2, 3), lambda b, n: (b, 0, 0)),
        ],
        out_specs=[
            pl.BlockSpec((1, TILE, 3), lambda b, n: (b, n, 0)),
            pl.BlockSpec((1, TILE, 3, 16), lambda b, n: (b, n, 0, 0)),
        ],
        out_shape=[
            jax.ShapeDtypeStruct((B, N1, 3), jnp.int32),
            jax.ShapeDtypeStruct((B, N1, 3, 16), jnp.float32),
        ],
    )(xyz1, xyz2)

    table = jnp.transpose(feats2, (0, 2, 1)).reshape(B * N2, C2)
    idx_flat = idx3.reshape(Q * 3)
    w_flat = w3.reshape(Q * 3, 16)

    mesh = plsc.VectorSubcoreMesh(core_axis_name="c", subcore_axis_name="s")
    sc_interp = functools.partial(
        pl.kernel,
        mesh=mesh,
        out_type=jax.ShapeDtypeStruct((Q, C2), jnp.float32),
        scratch_types=[
            pltpu.VMEM((CH * 3,), jnp.int32),
            pltpu.VMEM((CH * 3, 16), jnp.float32),
            pltpu.VMEM((CH * 3, C2), jnp.float32),
            pltpu.VMEM((CH, C2), jnp.float32),
            pltpu.SemaphoreType.DMA,
        ],
    )(_sc_interp_body)
    interp = sc_interp(table, idx_flat, w_flat)

    y, partials = pl.pallas_call(
        _passB_body,
        grid=(B, NT),
        in_specs=[
            pl.BlockSpec((1, TILE, C2), lambda b, n: (b, n, 0)),
            pl.BlockSpec((1, C1, TILE), lambda b, n: (b, 0, n)),
            pl.BlockSpec((OUT_CH, C2), lambda b, n: (0, 0)),
            pl.BlockSpec((OUT_CH, C1), lambda b, n: (0, 0)),
        ],
        out_specs=[
            pl.BlockSpec((1, OUT_CH, TILE), lambda b, n: (b, 0, n)),
            pl.BlockSpec((2, OUT_CH), lambda b, n: (0, 0)),
        ],
        out_shape=[
            jax.ShapeDtypeStruct((B, OUT_CH, N1), jnp.bfloat16),
            jax.ShapeDtypeStruct((2, OUT_CH), jnp.float32),
        ],
    )(interp.reshape(B, N1, C2), feats1, W[:, :C2], W[:, C2:])

    T2 = 2048
    out = pl.pallas_call(
        _pass2_body,
        grid=(B, N1 // T2),
        in_specs=[
            pl.BlockSpec((1, OUT_CH, T2), lambda b, n: (b, 0, n)),
            pl.BlockSpec((2, OUT_CH), lambda b, n: (0, 0)),
            pl.BlockSpec((1, OUT_CH), lambda b, n: (0, 0)),
            pl.BlockSpec((1, OUT_CH), lambda b, n: (0, 0)),
        ],
        out_specs=pl.BlockSpec((1, OUT_CH, T2), lambda b, n: (b, 0, n)),
        out_shape=jax.ShapeDtypeStruct((B, OUT_CH, N1), jnp.float32),
    )(y, partials, gamma.reshape(1, OUT_CH), beta.reshape(1, OUT_CH))
    return out


# speedup vs baseline: 2.3384x; 2.3384x over previous
import jax
import jax.numpy as jnp
from jax.experimental import pallas as pl

B, N1, N2 = 8, 8192, 2048
C1, C2 = 128, 256
IN_CH, OUT_CH = C1 + C2, 256
EPS_BN = 1e-5

TILE = 2048
NT = N1 // TILE


def _pass1_body(xyz1_ref, xyz2_ref, feats1_ref, feats2_ref, w_ref,
                y_ref, partial_ref):
    b = pl.program_id(0)
    n = pl.program_id(1)

    q = xyz1_ref[0]
    p = xyz2_ref[0]
    q2 = jnp.sum(q * q, axis=1, keepdims=True)
    p2 = jnp.sum(p * p, axis=1)
    qp = jax.lax.dot_general(q, p, (((1,), (1,)), ((), ())),
                             preferred_element_type=jnp.float32)
    d2 = q2 + p2[None, :] - 2.0 * qp

    work = d2
    s_mat = jnp.zeros((TILE, N2), jnp.float32)
    w_sum = jnp.zeros((TILE, 1), jnp.float32)
    for _ in range(3):
        mk = jnp.min(work, axis=1, keepdims=True)
        sel = work == mk
        dk = jnp.sqrt(jnp.maximum(mk, 1e-12)) + 1e-8
        wk = 1.0 / dk
        s_mat = jnp.where(sel, wk, s_mat)
        w_sum = w_sum + wk
        work = jnp.where(sel, jnp.float32(3.4e38), work)

    f2 = feats2_ref[0]
    interp = jax.lax.dot_general(f2, s_mat, (((1,), (1,)), ((), ())),
                                 preferred_element_type=jnp.float32)
    interp = interp * jnp.transpose(1.0 / w_sum, (1, 0))
    x = jnp.concatenate([interp, feats1_ref[0]], axis=0)
    y = jax.lax.dot_general(w_ref[...], x, (((1,), (0,)), ((), ())),
                            preferred_element_type=jnp.float32)
    y_ref[0] = y.astype(jnp.bfloat16)

    ps = jnp.sum(y, axis=1)
    pss = jnp.sum(y * y, axis=1)
    part = jnp.stack([ps, pss], axis=0)

    @pl.when(jnp.logical_and(b == 0, n == 0))
    def _init():
        partial_ref[...] = part

    @pl.when(jnp.logical_or(b != 0, n != 0))
    def _acc():
        partial_ref[...] = partial_ref[...] + part


def _pass2_body(y_ref, partial_ref, gamma_ref, beta_ref, out_ref):
    sums = partial_ref[...]
    cnt = jnp.float32(B * N1)
    mean = sums[0] / cnt
    var = sums[1] / cnt - mean * mean
    scale = gamma_ref[...][0] / jnp.sqrt(var + EPS_BN)
    shift = beta_ref[...][0] - mean * scale
    y = y_ref[0].astype(jnp.float32)
    out_ref[0] = jnp.maximum(y * scale[:, None] + shift[:, None], 0.0)


@jax.jit
def kernel(xyz1, xyz2, feats1, feats2, W, gamma, beta):
    y, partials = pl.pallas_call(
        _pass1_body,
        grid=(B, NT),
        in_specs=[
            pl.BlockSpec((1, TILE, 3), lambda b, n: (b, n, 0)),
            pl.BlockSpec((1, N2, 3), lambda b, n: (b, 0, 0)),
            pl.BlockSpec((1, C1, TILE), lambda b, n: (b, 0, n)),
            pl.BlockSpec((1, C2, N2), lambda b, n: (b, 0, 0)),
            pl.BlockSpec((OUT_CH, IN_CH), lambda b, n: (0, 0)),
        ],
        out_specs=[
            pl.BlockSpec((1, OUT_CH, TILE), lambda b, n: (b, 0, n)),
            pl.BlockSpec((2, OUT_CH), lambda b, n: (0, 0)),
        ],
        out_shape=[
            jax.ShapeDtypeStruct((B, OUT_CH, N1), jnp.bfloat16),
            jax.ShapeDtypeStruct((2, OUT_CH), jnp.float32),
        ],
    )(xyz1, xyz2, feats1, feats2, W)

    T2 = 8192
    out = pl.pallas_call(
        _pass2_body,
        grid=(B, N1 // T2),
        in_specs=[
            pl.BlockSpec((1, OUT_CH, T2), lambda b, n: (b, 0, n)),
            pl.BlockSpec((2, OUT_CH), lambda b, n: (0, 0)),
            pl.BlockSpec((1, OUT_CH), lambda b, n: (0, 0)),
            pl.BlockSpec((1, OUT_CH), lambda b, n: (0, 0)),
        ],
        out_specs=pl.BlockSpec((1, OUT_CH, T2), lambda b, n: (b, 0, n)),
        out_shape=jax.ShapeDtypeStruct((B, OUT_CH, N1), jnp.float32),
    )(y, partials, gamma.reshape(1, OUT_CH), beta.reshape(1, OUT_CH))
    return out


# speedup vs baseline: 2.4976x; 1.0681x over previous
import jax
import jax.numpy as jnp
from jax.experimental import pallas as pl

B, N1, N2 = 8, 8192, 2048
C1, C2 = 128, 256
IN_CH, OUT_CH = C1 + C2, 256
EPS_BN = 1e-5

TILE = 2048
NT = N1 // TILE


def _pass1_body(xyz1_ref, xyz2_ref, feats1_ref, feats2_ref, w_ref,
                y_ref, partial_ref):
    b = pl.program_id(0)
    n = pl.program_id(1)

    q = xyz1_ref[0]
    p = xyz2_ref[0]
    q2 = jnp.sum(q * q, axis=1, keepdims=True)
    p2 = jnp.sum(p * p, axis=1)
    qp2 = jax.lax.dot_general(q, -2.0 * p, (((1,), (1,)), ((), ())),
                              preferred_element_type=jnp.float32)
    d2 = (q2 + p2[None, :]) + qp2

    work = d2
    s_mat = jnp.zeros((TILE, N2), jnp.float32)
    w_sum = jnp.zeros((TILE, 1), jnp.float32)
    for _ in range(3):
        mk = jnp.min(work, axis=1, keepdims=True)
        sel = work == mk
        wk = jax.lax.rsqrt(jnp.maximum(mk, 1e-12))
        s_mat = jnp.where(sel, wk, s_mat)
        w_sum = w_sum + wk
        work = jnp.where(sel, jnp.float32(3.4e38), work)

    f2 = feats2_ref[0]
    interp = jax.lax.dot_general(f2, s_mat, (((1,), (1,)), ((), ())),
                                 preferred_element_type=jnp.float32)
    interp = interp * jnp.transpose(1.0 / w_sum, (1, 0))
    x = jnp.concatenate([interp, feats1_ref[0]], axis=0)
    y = jax.lax.dot_general(w_ref[...], x, (((1,), (0,)), ((), ())),
                            preferred_element_type=jnp.float32)
    y_ref[0] = y.astype(jnp.bfloat16)

    ps = jnp.sum(y, axis=1)
    pss = jnp.sum(y * y, axis=1)
    part = jnp.stack([ps, pss], axis=0)

    @pl.when(jnp.logical_and(b == 0, n == 0))
    def _init():
        partial_ref[...] = part

    @pl.when(jnp.logical_or(b != 0, n != 0))
    def _acc():
        partial_ref[...] = partial_ref[...] + part


def _pass2_body(y_ref, partial_ref, gamma_ref, beta_ref, out_ref):
    sums = partial_ref[...]
    cnt = jnp.float32(B * N1)
    mean = sums[0] / cnt
    var = sums[1] / cnt - mean * mean
    scale = gamma_ref[...][0] / jnp.sqrt(var + EPS_BN)
    shift = beta_ref[...][0] - mean * scale
    y = y_ref[0].astype(jnp.float32)
    out_ref[0] = jnp.maximum(y * scale[:, None] + shift[:, None], 0.0)


@jax.jit
def kernel(xyz1, xyz2, feats1, feats2, W, gamma, beta):
    y, partials = pl.pallas_call(
        _pass1_body,
        grid=(B, NT),
        in_specs=[
            pl.BlockSpec((1, TILE, 3), lambda b, n: (b, n, 0)),
            pl.BlockSpec((1, N2, 3), lambda b, n: (b, 0, 0)),
            pl.BlockSpec((1, C1, TILE), lambda b, n: (b, 0, n)),
            pl.BlockSpec((1, C2, N2), lambda b, n: (b, 0, 0)),
            pl.BlockSpec((OUT_CH, IN_CH), lambda b, n: (0, 0)),
        ],
        out_specs=[
            pl.BlockSpec((1, OUT_CH, TILE), lambda b, n: (b, 0, n)),
            pl.BlockSpec((2, OUT_CH), lambda b, n: (0, 0)),
        ],
        out_shape=[
            jax.ShapeDtypeStruct((B, OUT_CH, N1), jnp.bfloat16),
            jax.ShapeDtypeStruct((2, OUT_CH), jnp.float32),
        ],
    )(xyz1, xyz2, feats1, feats2, W)

    T2 = 8192
    out = pl.pallas_call(
        _pass2_body,
        grid=(B, N1 // T2),
        in_specs=[
            pl.BlockSpec((1, OUT_CH, T2), lambda b, n: (b, 0, n)),
            pl.BlockSpec((2, OUT_CH), lambda b, n: (0, 0)),
            pl.BlockSpec((1, OUT_CH), lambda b, n: (0, 0)),
            pl.BlockSpec((1, OUT_CH), lambda b, n: (0, 0)),
        ],
        out_specs=pl.BlockSpec((1, OUT_CH, T2), lambda b, n: (b, 0, n)),
        out_shape=jax.ShapeDtypeStruct((B, OUT_CH, N1), jnp.float32),
    )(y, partials, gamma.reshape(1, OUT_CH), beta.reshape(1, OUT_CH))
    return out
